# baseline (speedup 1.0000x reference)
import functools

import jax
import jax.numpy as jnp
from jax import lax
from jax.experimental import pallas as pl
from jax.experimental.pallas import tpu as pltpu
from jax.experimental.pallas import tpu_sc as plsc

_B, _S, _D = 1024, 200, 64
_V = 1000000
_NT = 20
_L = 16
_CH = 128
_NCH = _B // _CH


@functools.cache
def _build(nc: int, ns: int):
    nw = nc * ns
    mesh = plsc.VectorSubcoreMesh(
        core_axis_name="c", subcore_axis_name="s",
        num_cores=nc, num_subcores=ns)

    @functools.partial(
        pl.kernel,
        out_type=jax.ShapeDtypeStruct((_S, _D, _B), jnp.float32),
        mesh=mesh,
        scratch_types=[
            pltpu.VMEM((_B,), jnp.int32),
            pltpu.VMEM((_D, _B), jnp.float32),
            pltpu.VMEM((_NT, _D), jnp.float32),
            pltpu.SemaphoreType.DMA,
        ],
        compiler_params=pltpu.CompilerParams(use_tc_tiling_on_sc=False),
    )
    def soft_embed(tokt_hbm, wtet_hbm, learned_hbm, out_hbm,
                   tok_v, blk_v, lrn_v, sem):
        wid = lax.axis_index("s") * nc + lax.axis_index("c")
        lo = lax.shift_right_logical(wid * (_S // 8), 2)
        hi = lax.shift_right_logical((wid + 1) * (_S // 8), 2)
        pltpu.sync_copy(learned_hbm, lrn_v)

        def per_s(s, carry):
            @pl.when(s >= _NT)
            def _gather():
                pltpu.sync_copy(tokt_hbm.at[s], tok_v)
                cps = []
                for d in range(_D):
                    for ch in range(_NCH):
                        cps.append(pltpu.async_copy(
                            wtet_hbm.at[d].at[tok_v.at[pl.ds(ch * _CH, _CH)]],
                            blk_v.at[d].at[pl.ds(ch * _CH, _CH)], sem))
                for cp in cps:
                    cp.wait()

            @pl.when(s < _NT)
            def _learned():
                def fill_row(q, carry2):
                    for c in range(_D // _L):
                        v = lrn_v[s, pl.ds(c * _L, _L)]
                        for e in range(_L):
                            blk_v[c * _L + e, pl.ds(q * _L, _L)] = (
                                lax.broadcast(v[e], (_L,)))
                    return carry2
                lax.fori_loop(0, _B // _L, fill_row, 0)

            pltpu.sync_copy(blk_v, out_hbm.at[s])
            return carry

        lax.fori_loop(lo, hi, per_s, 0)

    return soft_embed


def kernel(tokens, wte_weight, learned_embedding):
    info = plsc.get_sparse_core_info()
    k = _build(info.num_cores, info.num_subcores)
    out_phys = k(tokens.astype(jnp.int32).T,
                 wte_weight.T,
                 learned_embedding)
    return out_phys.transpose(2, 0, 1)
